# baseline (speedup 1.0000x reference)
import functools

import jax
import jax.numpy as jnp
from jax import lax
from jax.experimental import pallas as pl
from jax.experimental.pallas import tpu as pltpu
from jax.experimental.pallas import tpu_sc as plsc

_ROWS_PER_BLOCK = 512


def _matvec_body(w_ref, emb_ref, v_ref, l1_ref):
    v_ref[...] = jnp.sum(emb_ref[...] * w_ref[...], axis=1)

    @pl.when(pl.program_id(0) == 0)
    def _():
        l1_ref[...] = jnp.sum(jnp.abs(w_ref[...]), keepdims=True)


def _matvec(emb, w):
    V, D = emb.shape
    nb = V // _ROWS_PER_BLOCK
    return pl.pallas_call(
        _matvec_body,
        grid=(nb,),
        in_specs=[
            pl.BlockSpec((1, D), lambda i: (0, 0)),
            pl.BlockSpec((_ROWS_PER_BLOCK, D), lambda i: (i, 0)),
        ],
        out_specs=[
            pl.BlockSpec((_ROWS_PER_BLOCK,), lambda i: (i,)),
            pl.BlockSpec((1, 1), lambda i: (0, 0)),
        ],
        out_shape=[
            jax.ShapeDtypeStruct((V,), jnp.float32),
            jax.ShapeDtypeStruct((1, 1), jnp.float32),
        ],
        compiler_params=pltpu.CompilerParams(
            dimension_semantics=("arbitrary",)
        ),
    )(w, emb)


_IDX_ROW = 128


@functools.lru_cache(maxsize=None)
def _make_gather(B, V):
    info = plsc.get_sparse_core_info()
    NC, NS = info.num_cores, info.num_subcores
    NW = NC * NS
    bpw = B // NW
    kj = bpw // _IDX_ROW
    mesh = plsc.VectorSubcoreMesh(core_axis_name="c", subcore_axis_name="s")

    @functools.partial(
        pl.kernel,
        mesh=mesh,
        out_type=jax.ShapeDtypeStruct((NW, kj, _IDX_ROW), jnp.float32),
        scratch_types=[
            pltpu.VMEM((kj, _IDX_ROW), jnp.int32),
            pltpu.VMEM((kj, _IDX_ROW), jnp.float32),
            pltpu.SemaphoreType.DMA,
        ],
    )
    def gather_k(v_hbm, x_hbm, out_hbm, idx_v, out_v, sem):
        wid = lax.axis_index("s") * NC + lax.axis_index("c")
        pltpu.sync_copy(x_hbm.at[wid], idx_v)
        copies = [
            pltpu.async_copy(v_hbm.at[idx_v.at[j]], out_v.at[j], sem)
            for j in range(kj)
        ]
        for c in copies:
            c.wait()
        pltpu.sync_copy(out_v, out_hbm.at[wid])

    return gather_k, NW, kj


_GB = 512


def _tc_gather_body(v_ref, x_ref, out_ref):
    v2 = v_ref[...].reshape(32, 128)
    xb = x_ref[0]
    hi = xb >> 7
    lo = xb & 127
    oh = (
        lax.broadcasted_iota(jnp.int32, (_GB, 32), 1) == hi
    ).astype(jnp.float32)
    r = jnp.dot(oh, v2, preferred_element_type=jnp.float32)
    out_ref[...] = jnp.take_along_axis(
        r, lo, axis=1, mode="promise_in_bounds"
    )


def _tc_gather(v, x4, B, V):
    nb = B // _GB
    return pl.pallas_call(
        _tc_gather_body,
        grid=(nb,),
        in_specs=[
            pl.BlockSpec((V,), lambda j: (0,)),
            pl.BlockSpec((1, _GB, 1), lambda j: (j, 0, 0)),
        ],
        out_specs=pl.BlockSpec((_GB, 1), lambda j: (j, 0)),
        out_shape=jax.ShapeDtypeStruct((B, 1), jnp.float32),
        compiler_params=pltpu.CompilerParams(
            dimension_semantics=("arbitrary",)
        ),
    )(v, x4)


def kernel(x, emb, W):
    B = x.shape[0]
    V, D = emb.shape
    v, l1 = _matvec(emb, W)
    x4 = x.astype(jnp.int32).reshape(B // _GB, _GB, 1)
    out = _tc_gather(v, x4, B, V)
    return out, l1[0, 0]


# speedup vs baseline: 1.6132x; 1.6132x over previous
import functools

import jax
import jax.numpy as jnp
from jax import lax
from jax.experimental import pallas as pl
from jax.experimental.pallas import tpu as pltpu
from jax.experimental.pallas import tpu_sc as plsc

_ROWS_PER_BLOCK = 512


def _matvec_body(w_ref, emb_ref, v_ref, l1_ref):
    v_ref[...] = jnp.sum(emb_ref[...] * w_ref[...], axis=1)

    @pl.when(pl.program_id(0) == 0)
    def _():
        l1_ref[...] = jnp.sum(jnp.abs(w_ref[...]), keepdims=True)


def _matvec(emb, w):
    V, D = emb.shape
    nb = V // _ROWS_PER_BLOCK
    return pl.pallas_call(
        _matvec_body,
        grid=(nb,),
        in_specs=[
            pl.BlockSpec((1, D), lambda i: (0, 0)),
            pl.BlockSpec((_ROWS_PER_BLOCK, D), lambda i: (i, 0)),
        ],
        out_specs=[
            pl.BlockSpec((_ROWS_PER_BLOCK,), lambda i: (i,)),
            pl.BlockSpec((1, 1), lambda i: (0, 0)),
        ],
        out_shape=[
            jax.ShapeDtypeStruct((V,), jnp.float32),
            jax.ShapeDtypeStruct((1, 1), jnp.float32),
        ],
        compiler_params=pltpu.CompilerParams(
            dimension_semantics=("arbitrary",)
        ),
    )(w, emb)


_IDX_ROW = 128


@functools.lru_cache(maxsize=None)
def _make_gather(B, V):
    info = plsc.get_sparse_core_info()
    NC, NS = info.num_cores, info.num_subcores
    NW = NC * NS
    bpw = B // NW
    kj = bpw // _IDX_ROW
    mesh = plsc.VectorSubcoreMesh(core_axis_name="c", subcore_axis_name="s")

    @functools.partial(
        pl.kernel,
        mesh=mesh,
        out_type=jax.ShapeDtypeStruct((NW, kj, _IDX_ROW), jnp.float32),
        scratch_types=[
            pltpu.VMEM((kj, _IDX_ROW), jnp.int32),
            pltpu.VMEM((kj, _IDX_ROW), jnp.float32),
            pltpu.SemaphoreType.DMA,
        ],
    )
    def gather_k(v_hbm, x_hbm, out_hbm, idx_v, out_v, sem):
        wid = lax.axis_index("s") * NC + lax.axis_index("c")
        pltpu.sync_copy(x_hbm.at[wid], idx_v)
        copies = [
            pltpu.async_copy(v_hbm.at[idx_v.at[j]], out_v.at[j], sem)
            for j in range(kj)
        ]
        for c in copies:
            c.wait()
        pltpu.sync_copy(out_v, out_hbm.at[wid])

    return gather_k, NW, kj


_GB = 512


def _tc_gather_body(v_ref, x_ref, out_ref):
    v2 = v_ref[...].reshape(32, 128)
    xb = x_ref[0]
    hi = xb >> 7
    lo = xb & 127
    oh = (
        lax.broadcasted_iota(jnp.int32, (32, _GB), 0) == hi
    ).astype(jnp.float32)
    t = lax.dot_general(
        v2, oh, (((0,), (0,)), ((), ())),
        preferred_element_type=jnp.float32,
    )
    lomask = lax.broadcasted_iota(jnp.int32, (128, _GB), 0) == lo
    g = jnp.sum(jnp.where(lomask, t, 0.0), axis=0, keepdims=True)
    out_ref[0] = g


def _tc_gather(v, x3, B, V):
    nb = B // _GB
    return pl.pallas_call(
        _tc_gather_body,
        grid=(nb,),
        in_specs=[
            pl.BlockSpec((V,), lambda j: (0,)),
            pl.BlockSpec((1, 1, _GB), lambda j: (j, 0, 0)),
        ],
        out_specs=pl.BlockSpec((1, 1, _GB), lambda j: (j, 0, 0)),
        out_shape=jax.ShapeDtypeStruct((nb, 1, _GB), jnp.float32),
        compiler_params=pltpu.CompilerParams(
            dimension_semantics=("arbitrary",)
        ),
    )(v, x3)


def kernel(x, emb, W):
    B = x.shape[0]
    V, D = emb.shape
    v, l1 = _matvec(emb, W)
    x3 = x.astype(jnp.int32).reshape(B // _GB, 1, _GB)
    out = _tc_gather(v, x3, B, V)
    return out.reshape(B, 1), l1[0, 0]
